# baseline (speedup 1.0000x reference)
import functools

import jax
import jax.numpy as jnp
from jax import lax
from jax.experimental import pallas as pl
from jax.experimental.pallas import tpu as pltpu
from jax.experimental.pallas import tpu_sc as plsc
import jax.experimental.layout as jlayout

V = 1000
N = 1024 * 50
NC, NS, L = 2, 16, 16
NW = NC * NS
PW = N // NW
CH = 32
NCH = PW // CH


def _lse_body(table_ref, lse_ref):
    x = table_ref[...]
    m = jnp.max(x, axis=1, keepdims=True)
    s = jnp.sum(jnp.exp(x - m), axis=1, keepdims=True)
    lse_ref[...] = (m + jnp.log(s))[:, 0]


_lse_call = pl.pallas_call(
    _lse_body,
    out_shape=jax.ShapeDtypeStruct((V,), jnp.float32),
)


_mesh = plsc.VectorSubcoreMesh(core_axis_name="c", subcore_axis_name="s")


@functools.partial(
    pl.kernel,
    out_type=[
        jax.ShapeDtypeStruct((N, V), jnp.float32),
        jax.ShapeDtypeStruct((NW, L), jnp.float32),
    ],
    mesh=_mesh,
    scratch_types=[
        pltpu.VMEM((CH,), jnp.int32),
        pltpu.VMEM((CH,), jnp.int32),
        pltpu.VMEM((CH,), jnp.int32),
        pltpu.VMEM((CH,), jnp.float32),
        pltpu.VMEM((CH,), jnp.float32),
        pltpu.VMEM((CH, V), jnp.float32),
        pltpu.VMEM((L,), jnp.float32),
        pltpu.SemaphoreType.DMA,
        pltpu.SemaphoreType.DMA,
    ],
    compiler_params=pltpu.CompilerParams(use_tc_tiling_on_sc=False),
)
def _sc_gather(idx_hbm, tgt_hbm, table_hbm, tableT_hbm, lse_hbm,
               out_hbm, part_hbm,
               idx_v, tgt_v, fidx_v, tval_v, lseg_v, rows_v, part_v,
               sem, sem2):
    wid = lax.axis_index("s") * NC + lax.axis_index("c")
    base = wid * PW

    def chunk(c, acc):
        off = base + c * CH
        pltpu.sync_copy(idx_hbm.at[pl.ds(off, CH)], idx_v)
        pltpu.sync_copy(tgt_hbm.at[pl.ds(off, CH)], tgt_v)
        row_dma = pltpu.async_copy(table_hbm.at[idx_v], rows_v, sem)
        lse_dma = pltpu.async_copy(lse_hbm.at[idx_v], lseg_v, sem2)
        for g in range(CH // L):
            i16 = idx_v[pl.ds(g * L, L)]
            t16 = tgt_v[pl.ds(g * L, L)]
            fidx_v[pl.ds(g * L, L)] = t16 * V + i16
        lse_dma.wait()
        pltpu.async_copy(tableT_hbm.at[fidx_v], tval_v, sem2).wait()
        for g in range(CH // L):
            acc = acc + (lseg_v[pl.ds(g * L, L)] - tval_v[pl.ds(g * L, L)])
        row_dma.wait()
        pltpu.sync_copy(rows_v, out_hbm.at[pl.ds(off, CH)])
        return acc

    acc = lax.fori_loop(0, NCH, chunk, jnp.zeros((L,), jnp.float32))
    part_v[...] = acc
    pltpu.sync_copy(part_v, part_hbm.at[wid])


def _loss_body(part_ref, loss_ref):
    loss_ref[0, 0] = jnp.sum(part_ref[...]) * (1.0 / N)


_loss_call = pl.pallas_call(
    _loss_body,
    out_shape=jax.ShapeDtypeStruct((1, 1), jnp.float32),
    out_specs=pl.BlockSpec(memory_space=pltpu.SMEM),
)


def _impl(idx, targets, table):
    idx_f = idx.reshape(N)
    tgt_f = targets.reshape(N)
    tableT_flat = table.T.reshape(V * V)
    lse = _lse_call(table)
    logits2, part = _sc_gather(idx_f, tgt_f, table, tableT_flat, lse)
    loss = _loss_call(part)[0, 0]
    return (logits2, loss)


_jitted = None
_jitted_plain = jax.jit(_impl)


def kernel(idx, targets, table):
    global _jitted
    if _jitted is None:
        try:
            dev = jax.devices("tpu")[0]
            fmt = jlayout.Format(
                jlayout.Layout((0, 1), tiling=((8,),)),
                jax.sharding.SingleDeviceSharding(dev),
            )
            _jitted = jax.jit(_impl, out_shardings=(fmt, None))
        except (RuntimeError, ValueError):
            _jitted = _jitted_plain
    try:
        return _jitted(idx, targets, table)
    except ValueError:
        return _jitted_plain(idx, targets, table)


# speedup vs baseline: 1.0020x; 1.0020x over previous
import functools

import jax
import jax.numpy as jnp
from jax import lax
from jax.experimental import pallas as pl
from jax.experimental.pallas import tpu as pltpu
from jax.experimental.pallas import tpu_sc as plsc
import jax.experimental.layout as jlayout

V = 1000
N = 1024 * 50
NC, NS, L = 2, 16, 16
NW = NC * NS
PW = N // NW
CH = 32
NCH = PW // CH


def _lse_body(table_ref, lse_ref):
    x = table_ref[...]
    m = jnp.max(x, axis=1, keepdims=True)
    s = jnp.sum(jnp.exp(x - m), axis=1, keepdims=True)
    lse_ref[...] = (m + jnp.log(s))[:, 0]


_lse_call = pl.pallas_call(
    _lse_body,
    out_shape=jax.ShapeDtypeStruct((V,), jnp.float32),
)


_mesh = plsc.VectorSubcoreMesh(core_axis_name="c", subcore_axis_name="s")


@functools.partial(
    pl.kernel,
    out_type=[
        jax.ShapeDtypeStruct((N, V), jnp.float32),
        jax.ShapeDtypeStruct((NW, L), jnp.float32),
    ],
    mesh=_mesh,
    scratch_types=[
        pltpu.VMEM((CH,), jnp.int32),
        pltpu.VMEM((CH,), jnp.int32),
        pltpu.VMEM((CH,), jnp.int32),
        pltpu.VMEM((CH,), jnp.float32),
        pltpu.VMEM((CH,), jnp.float32),
        pltpu.VMEM((CH, V), jnp.float32),
        pltpu.VMEM((L,), jnp.float32),
        pltpu.SemaphoreType.DMA,
        pltpu.SemaphoreType.DMA,
    ],
    compiler_params=pltpu.CompilerParams(use_tc_tiling_on_sc=False),
)
def _sc_gather(idx_hbm, tgt_hbm, table_hbm, tableT_hbm, lse_hbm,
               out_hbm, part_hbm,
               idx_v, tgt_v, fidx_v, tval_v, lseg_v, rows_v, part_v,
               sem, sem2):
    wid = lax.axis_index("s") * NC + lax.axis_index("c")
    base = wid * PW

    def chunk(c, acc):
        off = base + c * CH
        pltpu.sync_copy(idx_hbm.at[pl.ds(off, CH)], idx_v)
        pltpu.sync_copy(tgt_hbm.at[pl.ds(off, CH)], tgt_v)
        row_dma = pltpu.async_copy(table_hbm.at[idx_v], rows_v, sem)
        lse_dma = pltpu.async_copy(lse_hbm.at[idx_v], lseg_v, sem2)
        for g in range(CH // L):
            i16 = idx_v[pl.ds(g * L, L)]
            t16 = tgt_v[pl.ds(g * L, L)]
            fidx_v[pl.ds(g * L, L)] = t16 * V + i16
        lse_dma.wait()
        pltpu.async_copy(tableT_hbm.at[fidx_v], tval_v, sem2).wait()
        for g in range(CH // L):
            acc = acc + (lseg_v[pl.ds(g * L, L)] - tval_v[pl.ds(g * L, L)])
        row_dma.wait()
        pltpu.sync_copy(rows_v, out_hbm.at[pl.ds(off, CH)])
        return acc

    acc = lax.fori_loop(0, NCH, chunk, jnp.zeros((L,), jnp.float32))
    part_v[...] = acc
    pltpu.sync_copy(part_v, part_hbm.at[wid])


def _loss_body(part_ref, loss_ref):
    loss_ref[0, 0] = jnp.sum(part_ref[...]) * (1.0 / N)


_loss_call = pl.pallas_call(
    _loss_body,
    out_shape=jax.ShapeDtypeStruct((1, 1), jnp.float32),
    out_specs=pl.BlockSpec(memory_space=pltpu.SMEM),
)


def _impl(idx, targets, table):
    idx_f = idx.reshape(N)
    tgt_f = targets.reshape(N)
    tableT_flat = table.T.reshape(V * V)
    lse = _lse_call(table)
    logits2, part = _sc_gather(idx_f, tgt_f, table, tableT_flat, lse)
    loss = _loss_call(part)[0, 0]
    return (logits2, loss)


_jitted = None
_jitted_plain = jax.jit(_impl)


def kernel(idx, targets, table):
    global _jitted
    if _jitted is None:
        try:
            dev = jax.devices("tpu")[0]
            fmt = jlayout.Format(
                jlayout.Layout((0, 1), tiling=()),
                jax.sharding.SingleDeviceSharding(dev),
            )
            _jitted = jax.jit(_impl, out_shardings=(fmt, None))
        except (RuntimeError, ValueError):
            _jitted = _jitted_plain
    try:
        return _jitted(idx, targets, table)
    except ValueError:
        return _jitted_plain(idx, targets, table)


# speedup vs baseline: 2.2723x; 2.2678x over previous
import functools

import jax
import jax.numpy as jnp
from jax import lax
from jax.experimental import pallas as pl
from jax.experimental.pallas import tpu as pltpu
from jax.experimental.pallas import tpu_sc as plsc

V = 1000
N = 1024 * 50
NC, NS, L = 2, 16, 16
NW = NC * NS
PW = N // NW
BT = 512


def _prep_body(table_ref, lse_ref, hi_ref, lo_ref):
    x = table_ref[...]
    m = jnp.max(x, axis=1, keepdims=True)
    s = jnp.sum(jnp.exp(x - m), axis=1, keepdims=True)
    lse_ref[...] = (m + jnp.log(s))[:, 0]
    hi = x.astype(jnp.bfloat16)
    hi_ref[...] = hi
    lo_ref[...] = (x - hi.astype(jnp.float32)).astype(jnp.bfloat16)


_prep_call = pl.pallas_call(
    _prep_body,
    out_shape=[
        jax.ShapeDtypeStruct((V,), jnp.float32),
        jax.ShapeDtypeStruct((V, V), jnp.bfloat16),
        jax.ShapeDtypeStruct((V, V), jnp.bfloat16),
    ],
)


def _mm_body(idx_ref, hi_ref, lo_ref, out_ref):
    idxb = idx_ref[...]
    rows = lax.broadcasted_iota(jnp.int32, (V, BT), 0)
    oh = (rows == idxb[None, :]).astype(jnp.bfloat16)
    dn = (((0,), (0,)), ((), ()))
    acc = lax.dot_general(hi_ref[...], oh, dn,
                          preferred_element_type=jnp.float32)
    acc = acc + lax.dot_general(lo_ref[...], oh, dn,
                                preferred_element_type=jnp.float32)
    out_ref[...] = acc


_mm_call = pl.pallas_call(
    _mm_body,
    grid=(N // BT,),
    in_specs=[
        pl.BlockSpec((BT,), lambda i: (i,)),
        pl.BlockSpec((V, V), lambda i: (0, 0)),
        pl.BlockSpec((V, V), lambda i: (0, 0)),
    ],
    out_specs=pl.BlockSpec((V, BT), lambda i: (0, i)),
    out_shape=jax.ShapeDtypeStruct((V, N), jnp.float32),
)


_mesh = plsc.VectorSubcoreMesh(core_axis_name="c", subcore_axis_name="s")


@functools.partial(
    pl.kernel,
    out_type=jax.ShapeDtypeStruct((NW, L), jnp.float32),
    mesh=_mesh,
    scratch_types=[
        pltpu.VMEM((PW,), jnp.int32),
        pltpu.VMEM((PW,), jnp.int32),
        pltpu.VMEM((PW,), jnp.int32),
        pltpu.VMEM((PW,), jnp.float32),
        pltpu.VMEM((PW,), jnp.float32),
        pltpu.VMEM((L,), jnp.float32),
        pltpu.SemaphoreType.DMA,
        pltpu.SemaphoreType.DMA,
    ],
    compiler_params=pltpu.CompilerParams(use_tc_tiling_on_sc=False),
)
def _sc_loss(idx_hbm, tgt_hbm, tableT_hbm, lse_hbm, part_hbm,
             idx_v, tgt_v, fidx_v, tval_v, lseg_v, part_v, sem, sem2):
    wid = lax.axis_index("s") * NC + lax.axis_index("c")
    base = wid * PW

    pltpu.sync_copy(idx_hbm.at[pl.ds(base, PW)], idx_v)
    pltpu.sync_copy(tgt_hbm.at[pl.ds(base, PW)], tgt_v)
    lse_dma = pltpu.async_copy(lse_hbm.at[idx_v], lseg_v, sem)
    for g in range(PW // L):
        i16 = idx_v[pl.ds(g * L, L)]
        t16 = tgt_v[pl.ds(g * L, L)]
        fidx_v[pl.ds(g * L, L)] = t16 * V + i16
    pltpu.async_copy(tableT_hbm.at[fidx_v], tval_v, sem2).wait()
    lse_dma.wait()
    acc = jnp.zeros((L,), jnp.float32)
    for g in range(PW // L):
        acc = acc + (lseg_v[pl.ds(g * L, L)] - tval_v[pl.ds(g * L, L)])
    part_v[...] = acc
    pltpu.sync_copy(part_v, part_hbm.at[wid])


def _loss_body(part_ref, loss_ref):
    loss_ref[0, 0] = jnp.sum(part_ref[...]) * (1.0 / N)


_loss_call = pl.pallas_call(
    _loss_body,
    out_shape=jax.ShapeDtypeStruct((1, 1), jnp.float32),
    out_specs=pl.BlockSpec(memory_space=pltpu.SMEM),
)


def _impl(idx, targets, table):
    idx_f = idx.reshape(N)
    tgt_f = targets.reshape(N)
    tableT_flat = table.T.reshape(V * V)
    lse, hi, lo = _prep_call(table)
    part = _sc_loss(idx_f, tgt_f, tableT_flat, lse)
    loss = _loss_call(part)[0, 0]
    logitsT = _mm_call(idx_f, hi, lo)
    return (logitsT.T, loss)


_jitted = jax.jit(_impl)


def kernel(idx, targets, table):
    return _jitted(idx, targets, table)


# speedup vs baseline: 2.3636x; 1.0402x over previous
import functools

import jax
import jax.numpy as jnp
from jax import lax
from jax.experimental import pallas as pl
from jax.experimental.pallas import tpu as pltpu
from jax.experimental.pallas import tpu_sc as plsc

V = 1000
N = 1024 * 50
NC, NS, L = 2, 16, 16
NW = NC * NS
PW = N // NW
BT = 1024


def _prep_body(table_ref, tableT_ref, lse_ref, hi_ref, lo_ref):
    x = table_ref[...]
    m = jnp.max(x, axis=1, keepdims=True)
    s = jnp.sum(jnp.exp(x - m), axis=1, keepdims=True)
    lse_ref[...] = (m + jnp.log(s))[:, 0]
    xt = tableT_ref[...]
    hi = xt.astype(jnp.bfloat16)
    hi_ref[...] = hi
    lo_ref[...] = (xt - hi.astype(jnp.float32)).astype(jnp.bfloat16)


_prep_call = pl.pallas_call(
    _prep_body,
    out_shape=[
        jax.ShapeDtypeStruct((V,), jnp.float32),
        jax.ShapeDtypeStruct((V, V), jnp.bfloat16),
        jax.ShapeDtypeStruct((V, V), jnp.bfloat16),
    ],
)


def _mm_body(idx_ref, hi_ref, lo_ref, out_ref):
    idxb = idx_ref[...]
    rows = lax.broadcasted_iota(jnp.int32, (V, BT), 0)
    oh = (rows == idxb[None, :]).astype(jnp.bfloat16)
    dn = (((1,), (0,)), ((), ()))
    acc = lax.dot_general(hi_ref[...], oh, dn,
                          preferred_element_type=jnp.float32)
    acc = acc + lax.dot_general(lo_ref[...], oh, dn,
                                preferred_element_type=jnp.float32)
    out_ref[...] = acc


_mm_call = pl.pallas_call(
    _mm_body,
    grid=(N // BT,),
    in_specs=[
        pl.BlockSpec((BT,), lambda i: (i,)),
        pl.BlockSpec((V, V), lambda i: (0, 0)),
        pl.BlockSpec((V, V), lambda i: (0, 0)),
    ],
    out_specs=pl.BlockSpec((V, BT), lambda i: (0, i)),
    out_shape=jax.ShapeDtypeStruct((V, N), jnp.float32),
)


_mesh = plsc.VectorSubcoreMesh(core_axis_name="c", subcore_axis_name="s")


@functools.partial(
    pl.kernel,
    out_type=jax.ShapeDtypeStruct((NW, L), jnp.float32),
    mesh=_mesh,
    scratch_types=[
        pltpu.VMEM((PW,), jnp.int32),
        pltpu.VMEM((PW,), jnp.int32),
        pltpu.VMEM((PW,), jnp.int32),
        pltpu.VMEM((PW,), jnp.float32),
        pltpu.VMEM((PW,), jnp.float32),
        pltpu.VMEM((L,), jnp.float32),
        pltpu.SemaphoreType.DMA,
        pltpu.SemaphoreType.DMA,
    ],
    compiler_params=pltpu.CompilerParams(use_tc_tiling_on_sc=False),
)
def _sc_loss(idx_hbm, tgt_hbm, tableT_hbm, lse_hbm, part_hbm,
             idx_v, tgt_v, fidx_v, tval_v, lseg_v, part_v, sem, sem2):
    wid = lax.axis_index("s") * NC + lax.axis_index("c")
    base = wid * PW

    pltpu.sync_copy(idx_hbm.at[pl.ds(base, PW)], idx_v)
    pltpu.sync_copy(tgt_hbm.at[pl.ds(base, PW)], tgt_v)
    lse_dma = pltpu.async_copy(lse_hbm.at[idx_v], lseg_v, sem)
    for g in range(PW // L):
        i16 = idx_v[pl.ds(g * L, L)]
        t16 = tgt_v[pl.ds(g * L, L)]
        fidx_v[pl.ds(g * L, L)] = t16 * V + i16
    pltpu.async_copy(tableT_hbm.at[fidx_v], tval_v, sem2).wait()
    lse_dma.wait()
    acc = jnp.zeros((L,), jnp.float32)
    for g in range(PW // L):
        acc = acc + (lseg_v[pl.ds(g * L, L)] - tval_v[pl.ds(g * L, L)])
    part_v[...] = acc
    pltpu.sync_copy(part_v, part_hbm.at[wid])


def _loss_body(part_ref, loss_ref):
    loss_ref[0, 0] = jnp.sum(part_ref[...]) * (1.0 / N)


_loss_call = pl.pallas_call(
    _loss_body,
    out_shape=jax.ShapeDtypeStruct((1, 1), jnp.float32),
    out_specs=pl.BlockSpec(memory_space=pltpu.SMEM),
)


def _impl(idx, targets, table):
    idx_f = idx.reshape(N)
    tgt_f = targets.reshape(N)
    tableT = table.T
    tableT_flat = tableT.reshape(V * V)
    lse, hi, lo = _prep_call(table, tableT)
    part = _sc_loss(idx_f, tgt_f, tableT_flat, lse)
    loss = _loss_call(part)[0, 0]
    logitsT = _mm_call(idx_f, hi, lo)
    return (logitsT.T, loss)


_jitted = jax.jit(_impl)


def kernel(idx, targets, table):
    return _jitted(idx, targets, table)


# speedup vs baseline: 2.3645x; 1.0004x over previous
import functools

import jax
import jax.numpy as jnp
from jax import lax
from jax.experimental import pallas as pl
from jax.experimental.pallas import tpu as pltpu
from jax.experimental.pallas import tpu_sc as plsc

V = 1000
N = 1024 * 50
NC, NS, L = 2, 16, 16
NW = NC * NS
PW = N // NW
BT = 1024


def _prep_body(table_ref, tableT_ref, lse_ref, hi_ref, lo_ref):
    x = table_ref[...]
    m = jnp.max(x, axis=1, keepdims=True)
    s = jnp.sum(jnp.exp(x - m), axis=1, keepdims=True)
    lse_ref[...] = (m + jnp.log(s))[:, 0]
    xt = tableT_ref[...]
    hi = xt.astype(jnp.bfloat16)
    hi_ref[...] = hi
    lo_ref[...] = (xt - hi.astype(jnp.float32)).astype(jnp.bfloat16)


_prep_call = pl.pallas_call(
    _prep_body,
    out_shape=[
        jax.ShapeDtypeStruct((V,), jnp.float32),
        jax.ShapeDtypeStruct((V, V), jnp.bfloat16),
        jax.ShapeDtypeStruct((V, V), jnp.bfloat16),
    ],
)


def _mm_body(idx_ref, hi_ref, lo_ref, out_ref):
    idxb = idx_ref[...]
    rows = lax.broadcasted_iota(jnp.int32, (V, BT), 0)
    oh = (rows == idxb[None, :]).astype(jnp.bfloat16)
    dn = (((1,), (0,)), ((), ()))
    acc = lax.dot_general(hi_ref[...], oh, dn,
                          preferred_element_type=jnp.float32)
    acc = acc + lax.dot_general(lo_ref[...], oh, dn,
                                preferred_element_type=jnp.float32)
    out_ref[...] = acc


_mm_call = pl.pallas_call(
    _mm_body,
    grid=(N // BT,),
    in_specs=[
        pl.BlockSpec((BT,), lambda i: (i,)),
        pl.BlockSpec((V, V), lambda i: (0, 0)),
        pl.BlockSpec((V, V), lambda i: (0, 0)),
    ],
    out_specs=pl.BlockSpec((V, BT), lambda i: (0, i)),
    out_shape=jax.ShapeDtypeStruct((V, N), jnp.float32),
    compiler_params=pltpu.CompilerParams(
        dimension_semantics=("parallel",)),
)


_mesh = plsc.VectorSubcoreMesh(core_axis_name="c", subcore_axis_name="s")


@functools.partial(
    pl.kernel,
    out_type=jax.ShapeDtypeStruct((NW, L), jnp.float32),
    mesh=_mesh,
    scratch_types=[
        pltpu.VMEM((PW,), jnp.int32),
        pltpu.VMEM((PW,), jnp.int32),
        pltpu.VMEM((PW,), jnp.int32),
        pltpu.VMEM((PW,), jnp.float32),
        pltpu.VMEM((PW,), jnp.float32),
        pltpu.VMEM((L,), jnp.float32),
        pltpu.SemaphoreType.DMA,
        pltpu.SemaphoreType.DMA,
    ],
    compiler_params=pltpu.CompilerParams(use_tc_tiling_on_sc=False),
)
def _sc_loss(idx_hbm, tgt_hbm, tableT_hbm, lse_hbm, part_hbm,
             idx_v, tgt_v, fidx_v, tval_v, lseg_v, part_v, sem, sem2):
    wid = lax.axis_index("s") * NC + lax.axis_index("c")
    base = wid * PW

    pltpu.sync_copy(idx_hbm.at[pl.ds(base, PW)], idx_v)
    pltpu.sync_copy(tgt_hbm.at[pl.ds(base, PW)], tgt_v)
    lse_dma = pltpu.async_copy(lse_hbm.at[idx_v], lseg_v, sem)
    for g in range(PW // L):
        i16 = idx_v[pl.ds(g * L, L)]
        t16 = tgt_v[pl.ds(g * L, L)]
        fidx_v[pl.ds(g * L, L)] = t16 * V + i16
    pltpu.async_copy(tableT_hbm.at[fidx_v], tval_v, sem2).wait()
    lse_dma.wait()
    acc = jnp.zeros((L,), jnp.float32)
    for g in range(PW // L):
        acc = acc + (lseg_v[pl.ds(g * L, L)] - tval_v[pl.ds(g * L, L)])
    part_v[...] = acc
    pltpu.sync_copy(part_v, part_hbm.at[wid])


def _loss_body(part_ref, loss_ref):
    loss_ref[0, 0] = jnp.sum(part_ref[...]) * (1.0 / N)


_loss_call = pl.pallas_call(
    _loss_body,
    out_shape=jax.ShapeDtypeStruct((1, 1), jnp.float32),
    out_specs=pl.BlockSpec(memory_space=pltpu.SMEM),
)


def _impl(idx, targets, table):
    idx_f = idx.reshape(N)
    tgt_f = targets.reshape(N)
    tableT = table.T
    tableT_flat = tableT.reshape(V * V)
    lse, hi, lo = _prep_call(table, tableT)
    part = _sc_loss(idx_f, tgt_f, tableT_flat, lse)
    loss = _loss_call(part)[0, 0]
    logitsT = _mm_call(idx_f, hi, lo)
    return (logitsT.T, loss)


_jitted = jax.jit(_impl)


def kernel(idx, targets, table):
    return _jitted(idx, targets, table)


# speedup vs baseline: 2.4027x; 1.0162x over previous
import functools

import jax
import jax.numpy as jnp
from jax import lax
from jax.experimental import pallas as pl
from jax.experimental.pallas import tpu as pltpu
from jax.experimental.pallas import tpu_sc as plsc

V = 1000
N = 1024 * 50
NC, NS, L = 2, 16, 16
NW = NC * NS
PW = N // NW
BT = 2048


def _prep_body(table_ref, tableT_ref, lse_ref, hi_ref, lo_ref):
    x = table_ref[...]
    m = jnp.max(x, axis=1, keepdims=True)
    s = jnp.sum(jnp.exp(x - m), axis=1, keepdims=True)
    lse_ref[...] = (m + jnp.log(s))[:, 0]
    xt = tableT_ref[...]
    hi = xt.astype(jnp.bfloat16)
    hi_ref[...] = hi
    lo_ref[...] = (xt - hi.astype(jnp.float32)).astype(jnp.bfloat16)


_prep_call = pl.pallas_call(
    _prep_body,
    out_shape=[
        jax.ShapeDtypeStruct((V,), jnp.float32),
        jax.ShapeDtypeStruct((V, V), jnp.bfloat16),
        jax.ShapeDtypeStruct((V, V), jnp.bfloat16),
    ],
)


def _mm_body(idx_ref, hi_ref, lo_ref, out_ref):
    idxb = idx_ref[...]
    rows = lax.broadcasted_iota(jnp.int32, (V, BT), 0)
    oh = (rows == idxb[None, :]).astype(jnp.bfloat16)
    dn = (((1,), (0,)), ((), ()))
    acc = lax.dot_general(hi_ref[...], oh, dn,
                          preferred_element_type=jnp.float32)
    acc = acc + lax.dot_general(lo_ref[...], oh, dn,
                                preferred_element_type=jnp.float32)
    out_ref[...] = acc


_mm_call = pl.pallas_call(
    _mm_body,
    grid=(N // BT,),
    in_specs=[
        pl.BlockSpec((BT,), lambda i: (i,)),
        pl.BlockSpec((V, V), lambda i: (0, 0)),
        pl.BlockSpec((V, V), lambda i: (0, 0)),
    ],
    out_specs=pl.BlockSpec((V, BT), lambda i: (0, i)),
    out_shape=jax.ShapeDtypeStruct((V, N), jnp.float32),
    compiler_params=pltpu.CompilerParams(
        dimension_semantics=("parallel",)),
)


_mesh = plsc.VectorSubcoreMesh(core_axis_name="c", subcore_axis_name="s")


@functools.partial(
    pl.kernel,
    out_type=jax.ShapeDtypeStruct((NW, L), jnp.float32),
    mesh=_mesh,
    scratch_types=[
        pltpu.VMEM((PW,), jnp.int32),
        pltpu.VMEM((PW,), jnp.int32),
        pltpu.VMEM((PW,), jnp.int32),
        pltpu.VMEM((PW,), jnp.float32),
        pltpu.VMEM((PW,), jnp.float32),
        pltpu.VMEM((L,), jnp.float32),
        pltpu.SemaphoreType.DMA,
        pltpu.SemaphoreType.DMA,
    ],
    compiler_params=pltpu.CompilerParams(use_tc_tiling_on_sc=False),
)
def _sc_loss(idx_hbm, tgt_hbm, tableT_hbm, lse_hbm, part_hbm,
             idx_v, tgt_v, fidx_v, tval_v, lseg_v, part_v, sem, sem2):
    wid = lax.axis_index("s") * NC + lax.axis_index("c")
    base = wid * PW

    pltpu.sync_copy(idx_hbm.at[pl.ds(base, PW)], idx_v)
    pltpu.sync_copy(tgt_hbm.at[pl.ds(base, PW)], tgt_v)
    lse_dma = pltpu.async_copy(lse_hbm.at[idx_v], lseg_v, sem)
    for g in range(PW // L):
        i16 = idx_v[pl.ds(g * L, L)]
        t16 = tgt_v[pl.ds(g * L, L)]
        fidx_v[pl.ds(g * L, L)] = t16 * V + i16
    pltpu.async_copy(tableT_hbm.at[fidx_v], tval_v, sem2).wait()
    lse_dma.wait()
    acc = jnp.zeros((L,), jnp.float32)
    for g in range(PW // L):
        acc = acc + (lseg_v[pl.ds(g * L, L)] - tval_v[pl.ds(g * L, L)])
    part_v[...] = acc
    pltpu.sync_copy(part_v, part_hbm.at[wid])


def _loss_body(part_ref, loss_ref):
    loss_ref[0, 0] = jnp.sum(part_ref[...]) * (1.0 / N)


_loss_call = pl.pallas_call(
    _loss_body,
    out_shape=jax.ShapeDtypeStruct((1, 1), jnp.float32),
    out_specs=pl.BlockSpec(memory_space=pltpu.SMEM),
)


def _impl(idx, targets, table):
    idx_f = idx.reshape(N)
    tgt_f = targets.reshape(N)
    tableT = table.T
    tableT_flat = tableT.reshape(V * V)
    lse, hi, lo = _prep_call(table, tableT)
    part = _sc_loss(idx_f, tgt_f, tableT_flat, lse)
    loss = _loss_call(part)[0, 0]
    logitsT = _mm_call(idx_f, hi, lo)
    return (logitsT.T, loss)


_jitted = jax.jit(_impl)


def kernel(idx, targets, table):
    return _jitted(idx, targets, table)


# speedup vs baseline: 2.4059x; 1.0013x over previous
import functools

import jax
import jax.numpy as jnp
from jax import lax
from jax.experimental import pallas as pl
from jax.experimental.pallas import tpu as pltpu
from jax.experimental.pallas import tpu_sc as plsc

V = 1000
N = 1024 * 50
NC, NS, L = 2, 16, 16
NW = NC * NS
PW = N // NW
BT = 2048


def _prep_body(table_ref, tableT_ref, lse_ref, hi_ref, lo_ref):
    x = table_ref[...]
    m = jnp.max(x, axis=1, keepdims=True)
    s = jnp.sum(jnp.exp(x - m), axis=1, keepdims=True)
    lse_ref[...] = (m + jnp.log(s))[:, 0]
    xt = tableT_ref[...]
    hi = xt.astype(jnp.bfloat16)
    hi_ref[...] = hi
    lo_ref[...] = (xt - hi.astype(jnp.float32)).astype(jnp.bfloat16)


_prep_call = pl.pallas_call(
    _prep_body,
    out_shape=[
        jax.ShapeDtypeStruct((V,), jnp.float32),
        jax.ShapeDtypeStruct((V, V), jnp.bfloat16),
        jax.ShapeDtypeStruct((V, V), jnp.bfloat16),
    ],
)


CW = 512


def _mm_body(idx_ref, hi_ref, lo_ref, out_ref):
    hi = hi_ref[...]
    lo = lo_ref[...]
    dn = (((1,), (0,)), ((), ()))
    for c in range(BT // CW):
        idxc = idx_ref[pl.ds(c * CW, CW)]
        rows = lax.broadcasted_iota(jnp.int32, (V, CW), 0)
        oh = (rows == idxc[None, :]).astype(jnp.bfloat16)
        acc = lax.dot_general(hi, oh, dn,
                              preferred_element_type=jnp.float32)
        acc = acc + lax.dot_general(lo, oh, dn,
                                    preferred_element_type=jnp.float32)
        out_ref[:, c * CW:(c + 1) * CW] = acc


_mm_call = pl.pallas_call(
    _mm_body,
    grid=(N // BT,),
    in_specs=[
        pl.BlockSpec((BT,), lambda i: (i,)),
        pl.BlockSpec((V, V), lambda i: (0, 0)),
        pl.BlockSpec((V, V), lambda i: (0, 0)),
    ],
    out_specs=pl.BlockSpec((V, BT), lambda i: (0, i)),
    out_shape=jax.ShapeDtypeStruct((V, N), jnp.float32),
    compiler_params=pltpu.CompilerParams(
        dimension_semantics=("parallel",)),
)


_mesh = plsc.VectorSubcoreMesh(core_axis_name="c", subcore_axis_name="s")


@functools.partial(
    pl.kernel,
    out_type=jax.ShapeDtypeStruct((NW, L), jnp.float32),
    mesh=_mesh,
    scratch_types=[
        pltpu.VMEM((PW,), jnp.int32),
        pltpu.VMEM((PW,), jnp.int32),
        pltpu.VMEM((PW,), jnp.int32),
        pltpu.VMEM((PW,), jnp.float32),
        pltpu.VMEM((PW,), jnp.float32),
        pltpu.VMEM((L,), jnp.float32),
        pltpu.SemaphoreType.DMA,
        pltpu.SemaphoreType.DMA,
    ],
    compiler_params=pltpu.CompilerParams(use_tc_tiling_on_sc=False),
)
def _sc_loss(idx_hbm, tgt_hbm, tableT_hbm, lse_hbm, part_hbm,
             idx_v, tgt_v, fidx_v, tval_v, lseg_v, part_v, sem, sem2):
    wid = lax.axis_index("s") * NC + lax.axis_index("c")
    base = wid * PW

    pltpu.sync_copy(idx_hbm.at[pl.ds(base, PW)], idx_v)
    pltpu.sync_copy(tgt_hbm.at[pl.ds(base, PW)], tgt_v)
    lse_dma = pltpu.async_copy(lse_hbm.at[idx_v], lseg_v, sem)
    for g in range(PW // L):
        i16 = idx_v[pl.ds(g * L, L)]
        t16 = tgt_v[pl.ds(g * L, L)]
        fidx_v[pl.ds(g * L, L)] = t16 * V + i16
    pltpu.async_copy(tableT_hbm.at[fidx_v], tval_v, sem2).wait()
    lse_dma.wait()
    acc = jnp.zeros((L,), jnp.float32)
    for g in range(PW // L):
        acc = acc + (lseg_v[pl.ds(g * L, L)] - tval_v[pl.ds(g * L, L)])
    part_v[...] = acc
    pltpu.sync_copy(part_v, part_hbm.at[wid])


def _loss_body(part_ref, loss_ref):
    loss_ref[0, 0] = jnp.sum(part_ref[...]) * (1.0 / N)


_loss_call = pl.pallas_call(
    _loss_body,
    out_shape=jax.ShapeDtypeStruct((1, 1), jnp.float32),
    out_specs=pl.BlockSpec(memory_space=pltpu.SMEM),
)


def _impl(idx, targets, table):
    idx_f = idx.reshape(N)
    tgt_f = targets.reshape(N)
    tableT = table.T
    tableT_flat = tableT.reshape(V * V)
    lse, hi, lo = _prep_call(table, tableT)
    part = _sc_loss(idx_f, tgt_f, tableT_flat, lse)
    loss = _loss_call(part)[0, 0]
    logitsT = _mm_call(idx_f, hi, lo)
    return (logitsT.T, loss)


_jitted = jax.jit(_impl)


def kernel(idx, targets, table):
    return _jitted(idx, targets, table)


# speedup vs baseline: 2.4072x; 1.0005x over previous
import functools

import jax
import jax.numpy as jnp
from jax import lax
from jax.experimental import pallas as pl
from jax.experimental.pallas import tpu as pltpu
from jax.experimental.pallas import tpu_sc as plsc

V = 1000
N = 1024 * 50
NC, NS, L = 2, 16, 16
NW = NC * NS
PW = N // NW
BT = 2048


def _prep_body(table_ref, lse_ref, tableT_ref, hi_ref, lo_ref):
    x = table_ref[...]
    m = jnp.max(x, axis=1, keepdims=True)
    s = jnp.sum(jnp.exp(x - m), axis=1, keepdims=True)
    lse_ref[...] = (m + jnp.log(s))[:, 0]
    xt = x.T
    tableT_ref[...] = xt
    hi = xt.astype(jnp.bfloat16)
    hi_ref[...] = hi
    lo_ref[...] = (xt - hi.astype(jnp.float32)).astype(jnp.bfloat16)


_prep_call = pl.pallas_call(
    _prep_body,
    out_shape=[
        jax.ShapeDtypeStruct((V,), jnp.float32),
        jax.ShapeDtypeStruct((V, V), jnp.float32),
        jax.ShapeDtypeStruct((V, V), jnp.bfloat16),
        jax.ShapeDtypeStruct((V, V), jnp.bfloat16),
    ],
)


CW = 512


def _mm_body(idx_ref, hi_ref, lo_ref, out_ref):
    hi = hi_ref[...]
    lo = lo_ref[...]
    dn = (((1,), (0,)), ((), ()))
    for c in range(BT // CW):
        idxc = idx_ref[pl.ds(c * CW, CW)]
        rows = lax.broadcasted_iota(jnp.int32, (V, CW), 0)
        oh = (rows == idxc[None, :]).astype(jnp.bfloat16)
        acc = lax.dot_general(hi, oh, dn,
                              preferred_element_type=jnp.float32)
        acc = acc + lax.dot_general(lo, oh, dn,
                                    preferred_element_type=jnp.float32)
        out_ref[:, c * CW:(c + 1) * CW] = acc


_mm_call = pl.pallas_call(
    _mm_body,
    grid=(N // BT,),
    in_specs=[
        pl.BlockSpec((BT,), lambda i: (i,)),
        pl.BlockSpec((V, V), lambda i: (0, 0)),
        pl.BlockSpec((V, V), lambda i: (0, 0)),
    ],
    out_specs=pl.BlockSpec((V, BT), lambda i: (0, i)),
    out_shape=jax.ShapeDtypeStruct((V, N), jnp.float32),
    compiler_params=pltpu.CompilerParams(
        dimension_semantics=("parallel",)),
)


_mesh = plsc.VectorSubcoreMesh(core_axis_name="c", subcore_axis_name="s")


@functools.partial(
    pl.kernel,
    out_type=jax.ShapeDtypeStruct((NW, L), jnp.float32),
    mesh=_mesh,
    scratch_types=[
        pltpu.VMEM((PW,), jnp.int32),
        pltpu.VMEM((PW,), jnp.int32),
        pltpu.VMEM((PW,), jnp.int32),
        pltpu.VMEM((PW,), jnp.float32),
        pltpu.VMEM((PW,), jnp.float32),
        pltpu.VMEM((L,), jnp.float32),
        pltpu.SemaphoreType.DMA,
        pltpu.SemaphoreType.DMA,
    ],
    compiler_params=pltpu.CompilerParams(use_tc_tiling_on_sc=False),
)
def _sc_loss(idx_hbm, tgt_hbm, tableT_hbm, lse_hbm, part_hbm,
             idx_v, tgt_v, fidx_v, tval_v, lseg_v, part_v, sem, sem2):
    wid = lax.axis_index("s") * NC + lax.axis_index("c")
    base = wid * PW

    pltpu.sync_copy(idx_hbm.at[pl.ds(base, PW)], idx_v)
    pltpu.sync_copy(tgt_hbm.at[pl.ds(base, PW)], tgt_v)
    lse_dma = pltpu.async_copy(lse_hbm.at[idx_v], lseg_v, sem)
    for g in range(PW // L):
        i16 = idx_v[pl.ds(g * L, L)]
        t16 = tgt_v[pl.ds(g * L, L)]
        fidx_v[pl.ds(g * L, L)] = t16 * V + i16
    pltpu.async_copy(tableT_hbm.at[fidx_v], tval_v, sem2).wait()
    lse_dma.wait()
    acc = jnp.zeros((L,), jnp.float32)
    for g in range(PW // L):
        acc = acc + (lseg_v[pl.ds(g * L, L)] - tval_v[pl.ds(g * L, L)])
    part_v[...] = acc
    pltpu.sync_copy(part_v, part_hbm.at[wid])


def _loss_body(part_ref, loss_ref):
    loss_ref[0, 0] = jnp.sum(part_ref[...]) * (1.0 / N)


_loss_call = pl.pallas_call(
    _loss_body,
    out_shape=jax.ShapeDtypeStruct((1, 1), jnp.float32),
    out_specs=pl.BlockSpec(memory_space=pltpu.SMEM),
)


def _impl(idx, targets, table):
    idx_f = idx.reshape(N)
    tgt_f = targets.reshape(N)
    lse, tableT, hi, lo = _prep_call(table)
    tableT_flat = tableT.reshape(V * V)
    part = _sc_loss(idx_f, tgt_f, tableT_flat, lse)
    loss = _loss_call(part)[0, 0]
    logitsT = _mm_call(idx_f, hi, lo)
    return (logitsT.T, loss)


_jitted = jax.jit(_impl)


def kernel(idx, targets, table):
    return _jitted(idx, targets, table)


# speedup vs baseline: 2.4863x; 1.0329x over previous
import functools

import jax
import jax.numpy as jnp
from jax import lax
from jax.experimental import pallas as pl
from jax.experimental.pallas import tpu as pltpu
from jax.experimental.pallas import tpu_sc as plsc

V = 1000
N = 1024 * 50
NC, NS, L = 2, 16, 16
NW = NC * NS
PW = N // NW
BT = 4096


def _prep_body(table_ref, lse_ref, tableT_ref, hi_ref, lo_ref):
    x = table_ref[...]
    m = jnp.max(x, axis=1, keepdims=True)
    s = jnp.sum(jnp.exp(x - m), axis=1, keepdims=True)
    lse_ref[...] = (m + jnp.log(s))[:, 0]
    xt = x.T
    tableT_ref[...] = xt
    hi = xt.astype(jnp.bfloat16)
    hi_ref[...] = hi
    lo_ref[...] = (xt - hi.astype(jnp.float32)).astype(jnp.bfloat16)


_prep_call = pl.pallas_call(
    _prep_body,
    out_shape=[
        jax.ShapeDtypeStruct((V,), jnp.float32),
        jax.ShapeDtypeStruct((V, V), jnp.float32),
        jax.ShapeDtypeStruct((V, V), jnp.bfloat16),
        jax.ShapeDtypeStruct((V, V), jnp.bfloat16),
    ],
)


CW = 512


def _mm_body(idx_ref, hi_ref, lo_ref, out_ref):
    hi = hi_ref[...]
    lo = lo_ref[...]
    dn = (((1,), (0,)), ((), ()))
    for c in range(BT // CW):
        idxc = idx_ref[pl.ds(c * CW, CW)]
        rows = lax.broadcasted_iota(jnp.int32, (V, CW), 0)
        oh = (rows == idxc[None, :]).astype(jnp.bfloat16)
        acc = lax.dot_general(hi, oh, dn,
                              preferred_element_type=jnp.float32)
        acc = acc + lax.dot_general(lo, oh, dn,
                                    preferred_element_type=jnp.float32)
        out_ref[:, c * CW:(c + 1) * CW] = acc


_mm_call = pl.pallas_call(
    _mm_body,
    grid=(N // BT,),
    in_specs=[
        pl.BlockSpec((BT,), lambda i: (i,)),
        pl.BlockSpec((V, V), lambda i: (0, 0)),
        pl.BlockSpec((V, V), lambda i: (0, 0)),
    ],
    out_specs=pl.BlockSpec((V, BT), lambda i: (0, i)),
    out_shape=jax.ShapeDtypeStruct((V, N), jnp.float32),
    compiler_params=pltpu.CompilerParams(
        dimension_semantics=("parallel",)),
)


_mesh = plsc.VectorSubcoreMesh(core_axis_name="c", subcore_axis_name="s")


@functools.partial(
    pl.kernel,
    out_type=jax.ShapeDtypeStruct((NW, L), jnp.float32),
    mesh=_mesh,
    scratch_types=[
        pltpu.VMEM((PW,), jnp.int32),
        pltpu.VMEM((PW,), jnp.int32),
        pltpu.VMEM((PW,), jnp.int32),
        pltpu.VMEM((PW,), jnp.float32),
        pltpu.VMEM((PW,), jnp.float32),
        pltpu.VMEM((L,), jnp.float32),
        pltpu.SemaphoreType.DMA,
        pltpu.SemaphoreType.DMA,
    ],
    compiler_params=pltpu.CompilerParams(use_tc_tiling_on_sc=False),
)
def _sc_loss(idx_hbm, tgt_hbm, tableT_hbm, lse_hbm, part_hbm,
             idx_v, tgt_v, fidx_v, tval_v, lseg_v, part_v, sem, sem2):
    wid = lax.axis_index("s") * NC + lax.axis_index("c")
    base = wid * PW

    pltpu.sync_copy(idx_hbm.at[pl.ds(base, PW)], idx_v)
    pltpu.sync_copy(tgt_hbm.at[pl.ds(base, PW)], tgt_v)
    lse_dma = pltpu.async_copy(lse_hbm.at[idx_v], lseg_v, sem)
    for g in range(PW // L):
        i16 = idx_v[pl.ds(g * L, L)]
        t16 = tgt_v[pl.ds(g * L, L)]
        fidx_v[pl.ds(g * L, L)] = t16 * V + i16
    pltpu.async_copy(tableT_hbm.at[fidx_v], tval_v, sem2).wait()
    lse_dma.wait()
    acc = jnp.zeros((L,), jnp.float32)
    for g in range(PW // L):
        acc = acc + (lseg_v[pl.ds(g * L, L)] - tval_v[pl.ds(g * L, L)])
    part_v[...] = acc
    pltpu.sync_copy(part_v, part_hbm.at[wid])


def _loss_body(part_ref, loss_ref):
    loss_ref[0, 0] = jnp.sum(part_ref[...]) * (1.0 / N)


_loss_call = pl.pallas_call(
    _loss_body,
    out_shape=jax.ShapeDtypeStruct((1, 1), jnp.float32),
    out_specs=pl.BlockSpec(memory_space=pltpu.SMEM),
)


def _impl(idx, targets, table):
    idx_f = idx.reshape(N)
    tgt_f = targets.reshape(N)
    lse, tableT, hi, lo = _prep_call(table)
    tableT_flat = tableT.reshape(V * V)
    part = _sc_loss(idx_f, tgt_f, tableT_flat, lse)
    loss = _loss_call(part)[0, 0]
    logitsT = _mm_call(idx_f, hi, lo)
    return (logitsT.T, loss)


_jitted = jax.jit(_impl)


def kernel(idx, targets, table):
    return _jitted(idx, targets, table)
